# Initial kernel scaffold; baseline (speedup 1.0000x reference)
import jax
import jax.numpy as jnp
from jax.experimental import pallas as pl


def kernel(x, batch, W_g1, b_g1, W_g2, b_g2, W_l, b_l):
    raise NotImplementedError("write your pallas kernel here")


# speedup vs baseline: 5.4072x; 5.4072x over previous
import functools

import jax
import jax.numpy as jnp
from jax import lax
from jax.experimental import pallas as pl
from jax.experimental.pallas import tpu as pltpu
from jax.experimental.pallas import tpu_sc as plsc

N = 100000
D = 128
H = 64
G = 512
DL = 64
B = 2000
NBLK = N // B
RS_PAD = 640
LD = 144
SEG_PER = 16
OUTW = 448


def _gelu(z):
    return 0.5 * z * (1.0 + lax.erf(z * 0.7071067811865476))


def _tc_body(batch_ref, x_ref, wg1_ref, bg1_ref, w2t_ref, wl_ref, bl_ref,
             local_ref, gate_ref, rs_ref, rs_scr):
    pid = pl.program_id(0)
    x = x_ref[...]
    h = _gelu(jnp.dot(x, wg1_ref[...], preferred_element_type=jnp.float32)
              + bg1_ref[...])
    gate = jnp.sum(h * w2t_ref[...], axis=1)
    gate_ref[...] = gate.reshape(1, 1, B)
    local_ref[...] = _gelu(
        jnp.dot(x, wl_ref[...], preferred_element_type=jnp.float32)
        + bl_ref[...])

    b = batch_ref[0, 0, :]
    git = lax.broadcasted_iota(jnp.int32, (G, B), 0)
    lt = (b[None, :] < git).astype(jnp.int32)
    cnt = jnp.sum(lt, axis=1)

    @pl.when(pid == 0)
    def _():
        rs_scr[...] = jnp.zeros((G,), jnp.int32)

    rs_scr[...] += cnt

    @pl.when(pid == NBLK - 1)
    def _():
        rs_ref[...] = jnp.concatenate(
            [rs_scr[...], jnp.full((RS_PAD - G,), N, jnp.int32)])


def _sc_body(x_hbm, local_hbm, gate_hbm, rs_hbm, out_hbm,
             rsb, xbuf, lbuf, gbuf, outb):
    c = lax.axis_index("c")
    s = lax.axis_index("s")
    wid = s * 2 + c
    base_seg = wid * SEG_PER
    pltpu.sync_copy(rs_hbm.at[pl.ds(base_seg, 32)], rsb)
    zero = jnp.zeros((16,), jnp.float32)

    def seg_body(j, _):
        rv = rsb[pl.ds(j, 16)]
        seg_s = rv[0]
        seg_e = rv[1]

        ninf = jnp.full((16,), -jnp.inf, jnp.float32)
        carry = ((seg_s, jnp.float32(-jnp.inf), zero)
                 + (zero,) * 8 + (ninf,) * 8 + (zero,) * 8 + (zero,) * 4)

        def chunk(_, cy):
            cs = cy[0]
            base = jnp.minimum((cs // 8) * 8, N - LD)
            o = cs - base
            take = jnp.maximum(jnp.minimum(seg_e - cs, LD - o), 0)
            pltpu.sync_copy(x_hbm.at[pl.ds(base, LD)], xbuf)
            pltpu.sync_copy(local_hbm.at[pl.ds(base, LD)], lbuf)
            pltpu.sync_copy(gate_hbm.at[pl.ds(base, LD)], gbuf.at[pl.ds(0, LD)])

            def row(r, rc):
                m = rc[0]
                idx = o + r
                g = gbuf[pl.ds(idx, 16)][0]
                m_new = jnp.maximum(m, g)
                scale = jnp.exp(jnp.broadcast_to(m - m_new, (16,)))
                ev = jnp.exp(jnp.broadcast_to(g - m_new, (16,)))
                esum = rc[1] * scale + ev
                xs = [xbuf[idx, pl.ds(16 * k, 16)] for k in range(8)]
                ls = [lbuf[idx, pl.ds(16 * k, 16)] for k in range(4)]
                sx = tuple(rc[2 + k] + xs[k] for k in range(8))
                mx = tuple(jnp.maximum(rc[10 + k], xs[k]) for k in range(8))
                ex = tuple(rc[18 + k] * scale + ev * xs[k] for k in range(8))
                lsm = tuple(rc[26 + k] + ls[k] for k in range(4))
                return (m_new, esum) + sx + mx + ex + lsm

            rc = lax.fori_loop(0, take, row, cy[1:])
            return (cs + take,) + rc

        nch = (seg_e - seg_s + LD - 1) // LD + 1
        fin = lax.fori_loop(0, nch, chunk, carry)
        cntf = jnp.maximum((seg_e - seg_s).astype(jnp.float32), 1.0)
        inv = 1.0 / jnp.broadcast_to(cntf, (16,))
        has = seg_e > seg_s
        denom = jnp.where(has, fin[2], jnp.ones((16,), jnp.float32))
        inva = 1.0 / denom
        for k in range(8):
            outb[j, pl.ds(16 * k, 16)] = fin[3 + k] * inv
        for k in range(8):
            outb[j, pl.ds(128 + 16 * k, 16)] = fin[11 + k]
        for k in range(8):
            outb[j, pl.ds(256 + 16 * k, 16)] = fin[19 + k] * inva
        for k in range(4):
            outb[j, pl.ds(384 + 16 * k, 16)] = fin[27 + k] * inv
        return 0

    lax.fori_loop(0, SEG_PER, seg_body, 0)
    pltpu.sync_copy(outb, out_hbm.at[pl.ds(base_seg, SEG_PER)])


def kernel(x, batch, W_g1, b_g1, W_g2, b_g2, W_l, b_l):
    del b_g2
    batch3 = batch.astype(jnp.int32).reshape(NBLK, 1, B)
    w2t = W_g2.reshape(1, H)
    bg1 = b_g1.reshape(1, H)
    bl = b_l.reshape(1, DL)

    local, gate3, rs = pl.pallas_call(
        _tc_body,
        grid=(NBLK,),
        in_specs=[
            pl.BlockSpec((1, 1, B), lambda i: (i, 0, 0)),
            pl.BlockSpec((B, D), lambda i: (i, 0)),
            pl.BlockSpec((D, H), lambda i: (0, 0)),
            pl.BlockSpec((1, H), lambda i: (0, 0)),
            pl.BlockSpec((1, H), lambda i: (0, 0)),
            pl.BlockSpec((D, DL), lambda i: (0, 0)),
            pl.BlockSpec((1, DL), lambda i: (0, 0)),
        ],
        out_specs=[
            pl.BlockSpec((B, DL), lambda i: (i, 0)),
            pl.BlockSpec((1, 1, B), lambda i: (i, 0, 0)),
            pl.BlockSpec((RS_PAD,), lambda i: (0,)),
        ],
        out_shape=[
            jax.ShapeDtypeStruct((N, DL), jnp.float32),
            jax.ShapeDtypeStruct((NBLK, 1, B), jnp.float32),
            jax.ShapeDtypeStruct((RS_PAD,), jnp.int32),
        ],
        scratch_shapes=[pltpu.VMEM((G,), jnp.int32)],
    )(batch3, x, W_g1, bg1, w2t, W_l, bl)

    gate = gate3.reshape(N)

    out = pl.kernel(
        _sc_body,
        out_type=jax.ShapeDtypeStruct((G, OUTW), jnp.float32),
        mesh=plsc.VectorSubcoreMesh(core_axis_name="c", subcore_axis_name="s",
                                    num_cores=2, num_subcores=16),
        scratch_types=[
            pltpu.VMEM((32,), jnp.int32),
            pltpu.VMEM((LD, D), jnp.float32),
            pltpu.VMEM((LD, DL), jnp.float32),
            pltpu.VMEM((LD + 16,), jnp.float32),
            pltpu.VMEM((SEG_PER, OUTW), jnp.float32),
        ],
    )(x, local, gate, rs)
    return out


# speedup vs baseline: 10.0289x; 1.8547x over previous
import jax
import jax.numpy as jnp
from jax import lax
from jax.experimental import pallas as pl
from jax.experimental.pallas import tpu as pltpu
from jax.experimental.pallas import tpu_sc as plsc

N = 100000
D = 128
H = 64
G = 512
DL = 64
LGW = 80
B = 2000
NBLK = N // B
RS_PAD = 640
LD = 256
SEG_PER = 16
OUTW = 448


def _gelu(z):
    return 0.5 * z * (1.0 + lax.erf(z * 0.7071067811865476))


def _tc_body(batch_ref, x_ref, wg1_ref, bg1_ref, w2t_ref, wl_ref, bl_ref,
             lg_ref, rs_ref, rs_scr):
    pid = pl.program_id(0)
    x = x_ref[...]
    h = _gelu(jnp.dot(x, wg1_ref[...], preferred_element_type=jnp.float32)
              + bg1_ref[...])
    gate = jnp.sum(h * w2t_ref[...], axis=1, keepdims=True)
    loc = _gelu(jnp.dot(x, wl_ref[...], preferred_element_type=jnp.float32)
                + bl_ref[...])
    lg_ref[...] = jnp.concatenate(
        [loc, jnp.broadcast_to(gate, (B, LGW - DL))], axis=1)

    b = batch_ref[0, 0, :]
    git = lax.broadcasted_iota(jnp.int32, (G, B), 0)
    cnt = jnp.sum((b[None, :] < git).astype(jnp.int32), axis=1)

    @pl.when(pid == 0)
    def _():
        rs_scr[...] = jnp.zeros((G,), jnp.int32)

    rs_scr[...] += cnt

    @pl.when(pid == NBLK - 1)
    def _():
        rs_ref[...] = jnp.concatenate(
            [rs_scr[...], jnp.full((RS_PAD - G,), N, jnp.int32)])


def _sc_body(x_hbm, lg_hbm, rs_hbm, out_hbm,
             rsb, xb0, xb1, lb0, lb1, outb, sx0, sx1, sl0, sl1):
    c = lax.axis_index("c")
    s = lax.axis_index("s")
    wid = s * 2 + c
    base_seg = wid * SEG_PER
    pltpu.sync_copy(rs_hbm.at[pl.ds(base_seg, 32)], rsb)
    rv0 = rsb[pl.ds(0, 16)]
    rv1 = rsb[pl.ds(1, 16)]
    tile_s = rv0[0]
    tile_e = rv1[15]
    ts0 = (tile_s // 8) * 8
    nch = (tile_e - ts0 + LD - 1) // LD

    zero = jnp.zeros((16,), jnp.float32)
    ninf = jnp.full((16,), -jnp.inf, jnp.float32)

    def prefill(j, _):
        for k in range(8):
            outb[j, pl.ds(16 * k, 16)] = zero
        for k in range(8):
            outb[j, pl.ds(128 + 16 * k, 16)] = ninf
        for k in range(8):
            outb[j, pl.ds(256 + 16 * k, 16)] = zero
        for k in range(4):
            outb[j, pl.ds(384 + 16 * k, 16)] = zero
        return 0

    lax.fori_loop(0, SEG_PER, prefill, 0)

    def chunk_base(cix):
        return jnp.minimum(ts0 + cix * LD, N - LD)

    def start(cix, xb, lb, sx, sl):
        base = chunk_base(cix)
        pltpu.make_async_copy(x_hbm.at[pl.ds(base, LD)], xb, sx).start()
        pltpu.make_async_copy(
            lg_hbm.at[pl.ds(base * LGW, LD * LGW)], lb, sl).start()

    def wait(xb, lb, sx, sl):
        pltpu.make_async_copy(x_hbm.at[pl.ds(0, LD)], xb, sx).wait()
        pltpu.make_async_copy(
            lg_hbm.at[pl.ds(0, LD * LGW)], lb, sl).wait()

    init_carry = ((jnp.int32(0), jnp.int32(0), jnp.float32(-jnp.inf), zero)
                  + (zero,) * 8 + (ninf,) * 8 + (zero,) * 8 + (zero,) * 4)

    def rows(lo, hi, base, xb, lb, car):
        def row(r, rc):
            idx = r - base
            lrow = idx * LGW
            g = lb[pl.ds(lrow + DL, 16)][0]
            m = rc[1]
            m_new = jnp.maximum(m, g)
            scale = jnp.exp(jnp.broadcast_to(m - m_new, (16,)))
            ev = jnp.exp(jnp.broadcast_to(g - m_new, (16,)))
            esum = rc[2] * scale + ev
            xs = [xb[idx, pl.ds(16 * k, 16)] for k in range(8)]
            ls = [lb[pl.ds(lrow + 16 * k, 16)] for k in range(4)]
            sx_ = tuple(rc[3 + k] + xs[k] for k in range(8))
            mx_ = tuple(jnp.maximum(rc[11 + k], xs[k]) for k in range(8))
            ex_ = tuple(rc[19 + k] * scale + ev * xs[k] for k in range(8))
            ls_ = tuple(rc[27 + k] + ls[k] for k in range(4))
            return (rc[0] + 1, m_new, esum) + sx_ + mx_ + ex_ + ls_

        return lax.fori_loop(lo, hi, row, car)

    def flush(jj, rc):
        cnt = rc[0]
        cntf = jnp.maximum(cnt.astype(jnp.float32), 1.0)
        inv = 1.0 / jnp.broadcast_to(cntf, (16,))
        denom = jnp.where(cnt > 0, rc[2], jnp.ones((16,), jnp.float32))
        inva = 1.0 / denom
        for k in range(8):
            outb[jj, pl.ds(16 * k, 16)] = rc[3 + k] * inv
        for k in range(8):
            outb[jj, pl.ds(128 + 16 * k, 16)] = rc[11 + k]
        for k in range(8):
            outb[jj, pl.ds(256 + 16 * k, 16)] = rc[19 + k] * inva
        for k in range(4):
            outb[jj, pl.ds(384 + 16 * k, 16)] = rc[27 + k] * inv
        return (jnp.int32(0), jnp.float32(-jnp.inf), zero) \
            + (zero,) * 8 + (ninf,) * 8 + (zero,) * 8 + (zero,) * 4

    def process(cix, xb, lb, car):
        base = chunk_base(cix)
        lo_c = jnp.minimum(jnp.maximum(tile_s, ts0 + cix * LD), tile_e)
        hi_c = jnp.minimum(tile_e, ts0 + (cix + 1) * LD)
        hi_c = jnp.maximum(hi_c, lo_c)
        hival = hi_c - 1
        j_end = jnp.int32(0)
        for k in range(16):
            j_end = j_end + (rv1[k] <= hival).astype(jnp.int32)
        j_cur = car[0]

        def jbody(jj, rc):
            rvj = rsb[pl.ds(jj, 16)]
            lo = jnp.maximum(rvj[0], lo_c)
            hi = jnp.minimum(rvj[1], hi_c)
            rc = rows(lo, hi, base, xb, lb, rc)
            return flush(jj, rc)

        rc = lax.fori_loop(j_cur, j_end, jbody, car[1:])
        rvj = rsb[pl.ds(j_end, 16)]
        lo = jnp.maximum(rvj[0], lo_c)
        hi = jnp.minimum(rvj[1], hi_c)
        rc = rows(lo, hi, base, xb, lb, rc)
        return (j_end,) + rc

    start(0, xb0, lb0, sx0, sl0)
    nc2 = (nch + 1) // 2

    def c2body(c2, car):
        wait(xb0, lb0, sx0, sl0)
        start(2 * c2 + 1, xb1, lb1, sx1, sl1)
        car = process(2 * c2, xb0, lb0, car)
        wait(xb1, lb1, sx1, sl1)
        start(2 * c2 + 2, xb0, lb0, sx0, sl0)
        car = process(2 * c2 + 1, xb1, lb1, car)
        return car

    car = lax.fori_loop(0, nc2, c2body, init_carry)
    wait(xb0, lb0, sx0, sl0)
    flush(car[0], car[1:])
    pltpu.sync_copy(outb, out_hbm.at[pl.ds(base_seg, SEG_PER)])


def kernel(x, batch, W_g1, b_g1, W_g2, b_g2, W_l, b_l):
    del b_g2
    batch3 = batch.astype(jnp.int32).reshape(NBLK, 1, B)
    w2t = W_g2.reshape(1, H)
    bg1 = b_g1.reshape(1, H)
    bl = b_l.reshape(1, DL)

    lg, rs = pl.pallas_call(
        _tc_body,
        grid=(NBLK,),
        in_specs=[
            pl.BlockSpec((1, 1, B), lambda i: (i, 0, 0)),
            pl.BlockSpec((B, D), lambda i: (i, 0)),
            pl.BlockSpec((D, H), lambda i: (0, 0)),
            pl.BlockSpec((1, H), lambda i: (0, 0)),
            pl.BlockSpec((1, H), lambda i: (0, 0)),
            pl.BlockSpec((D, DL), lambda i: (0, 0)),
            pl.BlockSpec((1, DL), lambda i: (0, 0)),
        ],
        out_specs=[
            pl.BlockSpec((B, LGW), lambda i: (i, 0)),
            pl.BlockSpec((RS_PAD,), lambda i: (0,)),
        ],
        out_shape=[
            jax.ShapeDtypeStruct((N, LGW), jnp.float32),
            jax.ShapeDtypeStruct((RS_PAD,), jnp.int32),
        ],
        scratch_shapes=[pltpu.VMEM((G,), jnp.int32)],
    )(batch3, x, W_g1, bg1, w2t, W_l, bl)

    lg_flat = lg.reshape(N * LGW)

    out = pl.kernel(
        _sc_body,
        out_type=jax.ShapeDtypeStruct((G, OUTW), jnp.float32),
        mesh=plsc.VectorSubcoreMesh(core_axis_name="c", subcore_axis_name="s",
                                    num_cores=2, num_subcores=16),
        scratch_types=[
            pltpu.VMEM((32,), jnp.int32),
            pltpu.VMEM((LD, D), jnp.float32),
            pltpu.VMEM((LD, D), jnp.float32),
            pltpu.VMEM((LD * LGW,), jnp.float32),
            pltpu.VMEM((LD * LGW,), jnp.float32),
            pltpu.VMEM((SEG_PER, OUTW), jnp.float32),
            pltpu.SemaphoreType.DMA,
            pltpu.SemaphoreType.DMA,
            pltpu.SemaphoreType.DMA,
            pltpu.SemaphoreType.DMA,
        ],
    )(x, lg_flat, rs)
    return out


# speedup vs baseline: 15.2935x; 1.5249x over previous
import jax
import jax.numpy as jnp
from jax import lax
from jax.experimental import pallas as pl
from jax.experimental.pallas import tpu as pltpu
from jax.experimental.pallas import tpu_sc as plsc

N = 100000
D = 128
H = 64
G = 512
DL = 64
LGW = 128
B = 4000
NBLK = N // B
RS_PAD = 640
LD = 232
SEG_PER = 16
OUTW = 448


def _gelu(z):
    return 0.5 * z * (1.0 + lax.erf(z * 0.7071067811865476))


def _tc_body(batch_ref, x_ref, wg1_ref, bg1_ref, w2t_ref, wl_ref, bl_ref,
             lg_ref, rs_ref, rs_scr):
    pid = pl.program_id(0)
    x = x_ref[...]
    h = _gelu(jnp.dot(x, wg1_ref[...], preferred_element_type=jnp.float32)
              + bg1_ref[...])
    gate = jnp.sum(h * w2t_ref[...], axis=1, keepdims=True)
    loc = _gelu(jnp.dot(x, wl_ref[...], preferred_element_type=jnp.float32)
                + bl_ref[...])
    lg_ref[...] = jnp.concatenate(
        [loc, jnp.broadcast_to(gate, (B, 16)),
         jnp.zeros((B, LGW - DL - 16), jnp.float32)], axis=1)

    b = batch_ref[0, 0, :]
    git = lax.broadcasted_iota(jnp.int32, (G, B), 0)
    cnt = jnp.sum((b[None, :] < git).astype(jnp.int32), axis=1)

    @pl.when(pid == 0)
    def _():
        rs_scr[...] = jnp.zeros((G,), jnp.int32)

    rs_scr[...] += cnt

    @pl.when(pid == NBLK - 1)
    def _():
        rs_ref[...] = jnp.concatenate(
            [rs_scr[...], jnp.full((RS_PAD - G,), N, jnp.int32)])


def _sc_body(x_hbm, lg_hbm, rs_hbm, out_hbm,
             rsb, xb0, xb1, lb0, lb1, outb, sx0, sx1, sl0, sl1):
    c = lax.axis_index("c")
    s = lax.axis_index("s")
    wid = s * 2 + c
    base_seg = wid * SEG_PER
    pltpu.sync_copy(rs_hbm.at[pl.ds(base_seg, 32)], rsb)
    rv0 = rsb[pl.ds(0, 16)]
    rv1 = rsb[pl.ds(1, 16)]
    tile_s = rv0[0]
    tile_e = rv1[15]
    ts0 = (tile_s // 8) * 8
    nch = (tile_e - ts0 + LD - 1) // LD

    zero = jnp.zeros((16,), jnp.float32)
    ninf = jnp.full((16,), -jnp.inf, jnp.float32)

    def prefill(j, _):
        for k in range(8):
            outb[j, pl.ds(16 * k, 16)] = zero
        for k in range(8):
            outb[j, pl.ds(128 + 16 * k, 16)] = ninf
        for k in range(8):
            outb[j, pl.ds(256 + 16 * k, 16)] = zero
        for k in range(4):
            outb[j, pl.ds(384 + 16 * k, 16)] = zero
        return 0

    lax.fori_loop(0, SEG_PER, prefill, 0)

    def chunk_base(cix):
        return jnp.minimum(ts0 + cix * LD, N - LD)

    def start(cix, xb, lb, sx, sl):
        base = chunk_base(cix)
        pltpu.make_async_copy(x_hbm.at[pl.ds(base, LD)], xb, sx).start()
        pltpu.make_async_copy(lg_hbm.at[pl.ds(base, LD)], lb, sl).start()

    def wait(xb, lb, sx, sl):
        pltpu.make_async_copy(x_hbm.at[pl.ds(0, LD)], xb, sx).wait()
        pltpu.make_async_copy(lg_hbm.at[pl.ds(0, LD)], lb, sl).wait()

    init_carry = ((jnp.int32(0), jnp.int32(0), jnp.float32(-jnp.inf), zero)
                  + (zero,) * 8 + (ninf,) * 8 + (zero,) * 8 + (zero,) * 4)

    def rows(lo, hi, base, xb, lb, car):
        def row(r, rc):
            idx = r - base
            g = lb[idx, pl.ds(DL, 16)][0]
            m = rc[1]
            m_new = jnp.maximum(m, g)
            scale = jnp.exp(jnp.broadcast_to(m - m_new, (16,)))
            ev = jnp.exp(jnp.broadcast_to(g - m_new, (16,)))
            esum = rc[2] * scale + ev
            xs = [xb[idx, pl.ds(16 * k, 16)] for k in range(8)]
            ls = [lb[idx, pl.ds(16 * k, 16)] for k in range(4)]
            sx_ = tuple(rc[3 + k] + xs[k] for k in range(8))
            mx_ = tuple(jnp.maximum(rc[11 + k], xs[k]) for k in range(8))
            ex_ = tuple(rc[19 + k] * scale + ev * xs[k] for k in range(8))
            ls_ = tuple(rc[27 + k] + ls[k] for k in range(4))
            return (rc[0] + 1, m_new, esum) + sx_ + mx_ + ex_ + ls_

        return lax.fori_loop(lo, hi, row, car)

    def flush(jj, rc):
        cnt = rc[0]
        cntf = jnp.maximum(cnt.astype(jnp.float32), 1.0)
        inv = 1.0 / jnp.broadcast_to(cntf, (16,))
        denom = jnp.where(cnt > 0, rc[2], jnp.ones((16,), jnp.float32))
        inva = 1.0 / denom
        for k in range(8):
            outb[jj, pl.ds(16 * k, 16)] = rc[3 + k] * inv
        for k in range(8):
            outb[jj, pl.ds(128 + 16 * k, 16)] = rc[11 + k]
        for k in range(8):
            outb[jj, pl.ds(256 + 16 * k, 16)] = rc[19 + k] * inva
        for k in range(4):
            outb[jj, pl.ds(384 + 16 * k, 16)] = rc[27 + k] * inv
        return (jnp.int32(0), jnp.float32(-jnp.inf), zero) \
            + (zero,) * 8 + (ninf,) * 8 + (zero,) * 8 + (zero,) * 4

    def process(cix, xb, lb, car):
        base = chunk_base(cix)
        lo_c = jnp.minimum(jnp.maximum(tile_s, ts0 + cix * LD), tile_e)
        hi_c = jnp.minimum(tile_e, ts0 + (cix + 1) * LD)
        hi_c = jnp.maximum(hi_c, lo_c)
        hival = hi_c - 1
        j_end = jnp.int32(0)
        for k in range(16):
            j_end = j_end + (rv1[k] <= hival).astype(jnp.int32)
        j_cur = car[0]

        def jbody(jj, rc):
            rvj = rsb[pl.ds(jj, 16)]
            lo = jnp.maximum(rvj[0], lo_c)
            hi = jnp.minimum(rvj[1], hi_c)
            rc = rows(lo, hi, base, xb, lb, rc)
            return flush(jj, rc)

        rc = lax.fori_loop(j_cur, j_end, jbody, car[1:])
        rvj = rsb[pl.ds(j_end, 16)]
        lo = jnp.maximum(rvj[0], lo_c)
        hi = jnp.minimum(rvj[1], hi_c)
        rc = rows(lo, hi, base, xb, lb, rc)
        return (j_end,) + rc

    start(0, xb0, lb0, sx0, sl0)
    nc2 = (nch + 1) // 2

    def c2body(c2, car):
        wait(xb0, lb0, sx0, sl0)
        start(2 * c2 + 1, xb1, lb1, sx1, sl1)
        car = process(2 * c2, xb0, lb0, car)
        wait(xb1, lb1, sx1, sl1)
        start(2 * c2 + 2, xb0, lb0, sx0, sl0)
        car = process(2 * c2 + 1, xb1, lb1, car)
        return car

    car = lax.fori_loop(0, nc2, c2body, init_carry)
    wait(xb0, lb0, sx0, sl0)
    flush(car[0], car[1:])
    pltpu.sync_copy(outb, out_hbm.at[pl.ds(base_seg, SEG_PER)])


def kernel(x, batch, W_g1, b_g1, W_g2, b_g2, W_l, b_l):
    del b_g2
    batch3 = batch.astype(jnp.int32).reshape(NBLK, 1, B)
    w2t = W_g2.reshape(1, H)
    bg1 = b_g1.reshape(1, H)
    bl = b_l.reshape(1, DL)

    lg, rs = pl.pallas_call(
        _tc_body,
        grid=(NBLK,),
        in_specs=[
            pl.BlockSpec((1, 1, B), lambda i: (i, 0, 0)),
            pl.BlockSpec((B, D), lambda i: (i, 0)),
            pl.BlockSpec((D, H), lambda i: (0, 0)),
            pl.BlockSpec((1, H), lambda i: (0, 0)),
            pl.BlockSpec((1, H), lambda i: (0, 0)),
            pl.BlockSpec((D, DL), lambda i: (0, 0)),
            pl.BlockSpec((1, DL), lambda i: (0, 0)),
        ],
        out_specs=[
            pl.BlockSpec((B, LGW), lambda i: (i, 0)),
            pl.BlockSpec((RS_PAD,), lambda i: (0,)),
        ],
        out_shape=[
            jax.ShapeDtypeStruct((N, LGW), jnp.float32),
            jax.ShapeDtypeStruct((RS_PAD,), jnp.int32),
        ],
        scratch_shapes=[pltpu.VMEM((G,), jnp.int32)],
    )(batch3, x, W_g1, bg1, w2t, W_l, bl)

    out = pl.kernel(
        _sc_body,
        out_type=jax.ShapeDtypeStruct((G, OUTW), jnp.float32),
        mesh=plsc.VectorSubcoreMesh(core_axis_name="c", subcore_axis_name="s",
                                    num_cores=2, num_subcores=16),
        scratch_types=[
            pltpu.VMEM((32,), jnp.int32),
            pltpu.VMEM((LD, D), jnp.float32),
            pltpu.VMEM((LD, D), jnp.float32),
            pltpu.VMEM((LD, LGW), jnp.float32),
            pltpu.VMEM((LD, LGW), jnp.float32),
            pltpu.VMEM((SEG_PER, OUTW), jnp.float32),
            pltpu.SemaphoreType.DMA,
            pltpu.SemaphoreType.DMA,
            pltpu.SemaphoreType.DMA,
            pltpu.SemaphoreType.DMA,
        ],
    )(x, lg, rs)
    return out


# speedup vs baseline: 16.4295x; 1.0743x over previous
import jax
import jax.numpy as jnp
from jax import lax
from jax.experimental import pallas as pl
from jax.experimental.pallas import tpu as pltpu
from jax.experimental.pallas import tpu_sc as plsc

N = 100000
D = 128
H = 64
G = 512
DL = 64
LGW = 128
B = 4000
NBLK = N // B
RS_PAD = 640
RSW = 72
RSP = 576
LD = 232
SEG_PER = 16
OUTW = 448


def _gelu(z):
    return 0.5 * z * (1.0 + lax.erf(z * 0.7071067811865476))


def _tc_body(batch_ref, x_ref, wg1_ref, bg1_ref, w2t_ref, wl_ref, bl_ref,
             lg_ref, rs_ref, rs_scr):
    pid = pl.program_id(0)
    x = x_ref[...]
    h = _gelu(jnp.dot(x, wg1_ref[...], preferred_element_type=jnp.float32)
              + bg1_ref[...])
    gate = jnp.sum(h * w2t_ref[...], axis=1, keepdims=True)
    loc = _gelu(jnp.dot(x, wl_ref[...], preferred_element_type=jnp.float32)
                + bl_ref[...])
    lg_ref[...] = jnp.concatenate(
        [loc, jnp.broadcast_to(gate, (B, 16)),
         jnp.zeros((B, LGW - DL - 16), jnp.float32)], axis=1)

    b = batch_ref[0, 0, :]
    lo = batch_ref[0, 0, 0]
    hi = batch_ref[0, 0, B - 1]
    wbase = (lo // 8) * 8

    @pl.when(pid == 0)
    def _():
        rs_scr[...] = jnp.zeros((RSP, 1), jnp.int32)

    @pl.when(hi - lo <= RSW - 8)
    def _():
        wg = wbase + lax.broadcasted_iota(jnp.int32, (RSW, B), 0)
        cw = jnp.sum((b[None, :] < wg).astype(jnp.int32), axis=1,
                     keepdims=True)
        wg1 = wbase + lax.broadcasted_iota(jnp.int32, (RSW, 1), 0)
        cw = jnp.where(wg1 <= hi, cw, 0)
        rs_scr[pl.ds(wbase, RSW), :] += cw

    @pl.when(hi - lo > RSW - 8)
    def _():
        git = lax.broadcasted_iota(jnp.int32, (RSP, B), 0)
        cf = jnp.sum((b[None, :] < git).astype(jnp.int32), axis=1,
                     keepdims=True)
        git1 = lax.broadcasted_iota(jnp.int32, (RSP, 1), 0)
        rs_scr[...] += jnp.where(git1 <= hi, cf, 0)

    git1 = lax.broadcasted_iota(jnp.int32, (RSP, 1), 0)
    rs_scr[...] += jnp.where(git1 > hi, B, 0)

    @pl.when(pid == NBLK - 1)
    def _():
        rs_ref[...] = jnp.concatenate(
            [rs_scr[...][:G, 0], jnp.full((RS_PAD - G,), N, jnp.int32)])


def _sc_body(x_hbm, lg_hbm, rs_hbm, out_hbm,
             rsb, xb0, xb1, lb0, lb1, outb, sx0, sx1, sl0, sl1):
    c = lax.axis_index("c")
    s = lax.axis_index("s")
    wid = s * 2 + c
    base_seg = wid * SEG_PER
    pltpu.sync_copy(rs_hbm.at[pl.ds(base_seg, 32)], rsb)
    rv0 = rsb[pl.ds(0, 16)]
    rv1 = rsb[pl.ds(1, 16)]
    tile_s = rv0[0]
    tile_e = rv1[15]
    ts0 = (tile_s // 8) * 8
    nch = (tile_e - ts0 + LD - 1) // LD

    zero = jnp.zeros((16,), jnp.float32)
    ninf = jnp.full((16,), -jnp.inf, jnp.float32)

    def prefill(j, _):
        for k in range(8):
            outb[j, pl.ds(16 * k, 16)] = zero
        for k in range(8):
            outb[j, pl.ds(128 + 16 * k, 16)] = ninf
        for k in range(8):
            outb[j, pl.ds(256 + 16 * k, 16)] = zero
        for k in range(4):
            outb[j, pl.ds(384 + 16 * k, 16)] = zero
        return 0

    lax.fori_loop(0, SEG_PER, prefill, 0)

    def chunk_base(cix):
        return jnp.minimum(ts0 + cix * LD, N - LD)

    def start(cix, xb, lb, sx, sl):
        base = chunk_base(cix)
        pltpu.make_async_copy(x_hbm.at[pl.ds(base, LD)], xb, sx).start()
        pltpu.make_async_copy(lg_hbm.at[pl.ds(base, LD)], lb, sl).start()

    def wait(xb, lb, sx, sl):
        pltpu.make_async_copy(x_hbm.at[pl.ds(0, LD)], xb, sx).wait()
        pltpu.make_async_copy(lg_hbm.at[pl.ds(0, LD)], lb, sl).wait()

    init_carry = ((jnp.int32(0), jnp.int32(0), jnp.float32(-jnp.inf), zero)
                  + (zero,) * 8 + (ninf,) * 8 + (zero,) * 8 + (zero,) * 4)

    def rows(lo, hi, base, xb, lb, car):
        def row(r, rc):
            idx = r - base
            g = lb[idx, pl.ds(DL, 16)][0]
            m = rc[1]
            m_new = jnp.maximum(m, g)
            scale = jnp.exp(jnp.broadcast_to(m - m_new, (16,)))
            ev = jnp.exp(jnp.broadcast_to(g - m_new, (16,)))
            esum = rc[2] * scale + ev
            xs = [xb[idx, pl.ds(16 * k, 16)] for k in range(8)]
            ls = [lb[idx, pl.ds(16 * k, 16)] for k in range(4)]
            sx_ = tuple(rc[3 + k] + xs[k] for k in range(8))
            mx_ = tuple(jnp.maximum(rc[11 + k], xs[k]) for k in range(8))
            ex_ = tuple(rc[19 + k] * scale + ev * xs[k] for k in range(8))
            ls_ = tuple(rc[27 + k] + ls[k] for k in range(4))
            return (rc[0] + 1, m_new, esum) + sx_ + mx_ + ex_ + ls_

        return lax.fori_loop(lo, hi, row, car)

    def flush(jj, rc):
        cnt = rc[0]
        cntf = jnp.maximum(cnt.astype(jnp.float32), 1.0)
        inv = 1.0 / jnp.broadcast_to(cntf, (16,))
        denom = jnp.where(cnt > 0, rc[2], jnp.ones((16,), jnp.float32))
        inva = 1.0 / denom
        for k in range(8):
            outb[jj, pl.ds(16 * k, 16)] = rc[3 + k] * inv
        for k in range(8):
            outb[jj, pl.ds(128 + 16 * k, 16)] = rc[11 + k]
        for k in range(8):
            outb[jj, pl.ds(256 + 16 * k, 16)] = rc[19 + k] * inva
        for k in range(4):
            outb[jj, pl.ds(384 + 16 * k, 16)] = rc[27 + k] * inv
        return (jnp.int32(0), jnp.float32(-jnp.inf), zero) \
            + (zero,) * 8 + (ninf,) * 8 + (zero,) * 8 + (zero,) * 4

    def process(cix, xb, lb, car):
        base = chunk_base(cix)
        lo_c = jnp.minimum(jnp.maximum(tile_s, ts0 + cix * LD), tile_e)
        hi_c = jnp.minimum(tile_e, ts0 + (cix + 1) * LD)
        hi_c = jnp.maximum(hi_c, lo_c)
        hival = hi_c - 1
        j_end = jnp.int32(0)
        for k in range(16):
            j_end = j_end + (rv1[k] <= hival).astype(jnp.int32)
        j_cur = car[0]

        def jbody(jj, rc):
            rvj = rsb[pl.ds(jj, 16)]
            lo = jnp.maximum(rvj[0], lo_c)
            hi = jnp.minimum(rvj[1], hi_c)
            rc = rows(lo, hi, base, xb, lb, rc)
            return flush(jj, rc)

        rc = lax.fori_loop(j_cur, j_end, jbody, car[1:])
        rvj = rsb[pl.ds(j_end, 16)]
        lo = jnp.maximum(rvj[0], lo_c)
        hi = jnp.minimum(rvj[1], hi_c)
        rc = rows(lo, hi, base, xb, lb, rc)
        return (j_end,) + rc

    start(0, xb0, lb0, sx0, sl0)
    nc2 = (nch + 1) // 2

    def c2body(c2, car):
        wait(xb0, lb0, sx0, sl0)
        start(2 * c2 + 1, xb1, lb1, sx1, sl1)
        car = process(2 * c2, xb0, lb0, car)
        wait(xb1, lb1, sx1, sl1)
        start(2 * c2 + 2, xb0, lb0, sx0, sl0)
        car = process(2 * c2 + 1, xb1, lb1, car)
        return car

    car = lax.fori_loop(0, nc2, c2body, init_carry)
    wait(xb0, lb0, sx0, sl0)
    flush(car[0], car[1:])
    pltpu.sync_copy(outb, out_hbm.at[pl.ds(base_seg, SEG_PER)])


def kernel(x, batch, W_g1, b_g1, W_g2, b_g2, W_l, b_l):
    del b_g2
    batch3 = batch.astype(jnp.int32).reshape(NBLK, 1, B)
    w2t = W_g2.reshape(1, H)
    bg1 = b_g1.reshape(1, H)
    bl = b_l.reshape(1, DL)

    lg, rs = pl.pallas_call(
        _tc_body,
        grid=(NBLK,),
        in_specs=[
            pl.BlockSpec((1, 1, B), lambda i: (i, 0, 0)),
            pl.BlockSpec((B, D), lambda i: (i, 0)),
            pl.BlockSpec((D, H), lambda i: (0, 0)),
            pl.BlockSpec((1, H), lambda i: (0, 0)),
            pl.BlockSpec((1, H), lambda i: (0, 0)),
            pl.BlockSpec((D, DL), lambda i: (0, 0)),
            pl.BlockSpec((1, DL), lambda i: (0, 0)),
        ],
        out_specs=[
            pl.BlockSpec((B, LGW), lambda i: (i, 0)),
            pl.BlockSpec((RS_PAD,), lambda i: (0,)),
        ],
        out_shape=[
            jax.ShapeDtypeStruct((N, LGW), jnp.float32),
            jax.ShapeDtypeStruct((RS_PAD,), jnp.int32),
        ],
        scratch_shapes=[pltpu.VMEM((RSP, 1), jnp.int32)],
    )(batch3, x, W_g1, bg1, w2t, W_l, bl)

    out = pl.kernel(
        _sc_body,
        out_type=jax.ShapeDtypeStruct((G, OUTW), jnp.float32),
        mesh=plsc.VectorSubcoreMesh(core_axis_name="c", subcore_axis_name="s",
                                    num_cores=2, num_subcores=16),
        scratch_types=[
            pltpu.VMEM((32,), jnp.int32),
            pltpu.VMEM((LD, D), jnp.float32),
            pltpu.VMEM((LD, D), jnp.float32),
            pltpu.VMEM((LD, LGW), jnp.float32),
            pltpu.VMEM((LD, LGW), jnp.float32),
            pltpu.VMEM((SEG_PER, OUTW), jnp.float32),
            pltpu.SemaphoreType.DMA,
            pltpu.SemaphoreType.DMA,
            pltpu.SemaphoreType.DMA,
            pltpu.SemaphoreType.DMA,
        ],
    )(x, lg, rs)
    return out


# speedup vs baseline: 16.6028x; 1.0106x over previous
import jax
import jax.numpy as jnp
from jax import lax
from jax.experimental import pallas as pl
from jax.experimental.pallas import tpu as pltpu
from jax.experimental.pallas import tpu_sc as plsc

N = 100000
D = 128
H = 64
G = 512
DL = 64
LGW = 128
B = 4000
NBLK = N // B
RS_PAD = 640
RSW = 72
RSP = 576
LD = 232
SEG_PER = 16
OUTW = 448


def _gelu(z):
    return 0.5 * z * (1.0 + lax.erf(z * 0.7071067811865476))


def _tc_body(batch_ref, x_ref, wg1_ref, bg1_ref, w2t_ref, wl_ref, bl_ref,
             lg_ref, rs_ref, rs_scr):
    pid = pl.program_id(0)
    x = x_ref[...]
    h = _gelu(jnp.dot(x, wg1_ref[...], preferred_element_type=jnp.float32)
              + bg1_ref[...])
    gate = jnp.sum(h * w2t_ref[...], axis=1, keepdims=True)
    loc = _gelu(jnp.dot(x, wl_ref[...], preferred_element_type=jnp.float32)
                + bl_ref[...])
    lg_ref[...] = jnp.concatenate(
        [loc, jnp.broadcast_to(gate, (B, 16)),
         jnp.zeros((B, LGW - DL - 16), jnp.float32)], axis=1)

    b = batch_ref[0, 0, :]
    lo = batch_ref[0, 0, 0]
    hi = batch_ref[0, 0, B - 1]
    wbase = (lo // 8) * 8

    @pl.when(pid == 0)
    def _():
        rs_scr[...] = jnp.zeros((RSP, 1), jnp.int32)

    @pl.when(hi - lo <= RSW - 8)
    def _():
        wg = wbase + lax.broadcasted_iota(jnp.int32, (RSW, B), 0)
        cw = jnp.sum((b[None, :] < wg).astype(jnp.int32), axis=1,
                     keepdims=True)
        wg1 = wbase + lax.broadcasted_iota(jnp.int32, (RSW, 1), 0)
        cw = jnp.where(wg1 <= hi, cw, 0)
        rs_scr[pl.ds(wbase, RSW), :] += cw

    @pl.when(hi - lo > RSW - 8)
    def _():
        git = lax.broadcasted_iota(jnp.int32, (RSP, B), 0)
        cf = jnp.sum((b[None, :] < git).astype(jnp.int32), axis=1,
                     keepdims=True)
        git1 = lax.broadcasted_iota(jnp.int32, (RSP, 1), 0)
        rs_scr[...] += jnp.where(git1 <= hi, cf, 0)

    git1 = lax.broadcasted_iota(jnp.int32, (RSP, 1), 0)
    rs_scr[...] += jnp.where(git1 > hi, B, 0)

    @pl.when(pid == NBLK - 1)
    def _():
        rs_ref[...] = jnp.concatenate(
            [rs_scr[...][:G, 0], jnp.full((RS_PAD - G,), N, jnp.int32)])


def _sc_body(x_hbm, lg_hbm, rs_hbm, out_hbm,
             rsb, xb0, xb1, lb0, lb1, outb, sx0, sx1, sl0, sl1):
    c = lax.axis_index("c")
    s = lax.axis_index("s")
    wid = s * 2 + c
    base_seg = wid * SEG_PER
    pltpu.sync_copy(rs_hbm.at[pl.ds(base_seg, 32)], rsb)
    rv0 = rsb[pl.ds(0, 16)]
    rv1 = rsb[pl.ds(1, 16)]
    tile_s = rv0[0]
    tile_e = rv1[15]
    ts0 = (tile_s // 8) * 8
    nch = (tile_e - ts0 + LD - 1) // LD

    zero = jnp.zeros((16,), jnp.float32)
    ninf = jnp.full((16,), -jnp.inf, jnp.float32)

    def prefill(j, _):
        for k in range(8):
            outb[j, pl.ds(16 * k, 16)] = zero
        for k in range(8):
            outb[j, pl.ds(128 + 16 * k, 16)] = ninf
        for k in range(8):
            outb[j, pl.ds(256 + 16 * k, 16)] = zero
        for k in range(4):
            outb[j, pl.ds(384 + 16 * k, 16)] = zero
        return 0

    lax.fori_loop(0, SEG_PER, prefill, 0)

    def chunk_base(cix):
        return jnp.minimum(ts0 + cix * LD, N - LD)

    def start(cix, xb, lb, sx, sl):
        base = chunk_base(cix)
        pltpu.make_async_copy(x_hbm.at[pl.ds(base, LD)], xb, sx).start()
        pltpu.make_async_copy(lg_hbm.at[pl.ds(base, LD)], lb, sl).start()

    def wait(xb, lb, sx, sl):
        pltpu.make_async_copy(x_hbm.at[pl.ds(0, LD)], xb, sx).wait()
        pltpu.make_async_copy(lg_hbm.at[pl.ds(0, LD)], lb, sl).wait()

    init_carry = ((jnp.int32(0), jnp.int32(0), zero, zero)
                  + (zero,) * 8 + (ninf,) * 8 + (zero,) * 8 + (zero,) * 4)

    def rows(lo, hi, base, xb, lb, car):
        def row(r, rc):
            idx = r - base
            g = lb[idx, pl.ds(DL, 16)][0]
            m0 = jnp.where(rc[0] == 0, jnp.broadcast_to(g, (16,)), rc[1])
            ev = jnp.exp(jnp.broadcast_to(g, (16,)) - m0)
            esum = rc[2] + ev
            xs = [xb[idx, pl.ds(16 * k, 16)] for k in range(8)]
            ls = [lb[idx, pl.ds(16 * k, 16)] for k in range(4)]
            sx_ = tuple(rc[3 + k] + xs[k] for k in range(8))
            mx_ = tuple(jnp.maximum(rc[11 + k], xs[k]) for k in range(8))
            ex_ = tuple(rc[19 + k] + ev * xs[k] for k in range(8))
            ls_ = tuple(rc[27 + k] + ls[k] for k in range(4))
            return (rc[0] + 1, m0, esum) + sx_ + mx_ + ex_ + ls_

        return lax.fori_loop(lo, hi, row, car)

    def flush(jj, rc):
        cnt = rc[0]
        cntf = jnp.maximum(cnt.astype(jnp.float32), 1.0)
        inv = 1.0 / jnp.broadcast_to(cntf, (16,))
        denom = jnp.where(cnt > 0, rc[2], jnp.ones((16,), jnp.float32))
        inva = 1.0 / denom
        for k in range(8):
            outb[jj, pl.ds(16 * k, 16)] = rc[3 + k] * inv
        for k in range(8):
            outb[jj, pl.ds(128 + 16 * k, 16)] = rc[11 + k]
        for k in range(8):
            outb[jj, pl.ds(256 + 16 * k, 16)] = rc[19 + k] * inva
        for k in range(4):
            outb[jj, pl.ds(384 + 16 * k, 16)] = rc[27 + k] * inv
        return (jnp.int32(0), zero, zero) \
            + (zero,) * 8 + (ninf,) * 8 + (zero,) * 8 + (zero,) * 4

    def process(cix, xb, lb, car):
        base = chunk_base(cix)
        lo_c = jnp.minimum(jnp.maximum(tile_s, ts0 + cix * LD), tile_e)
        hi_c = jnp.minimum(tile_e, ts0 + (cix + 1) * LD)
        hi_c = jnp.maximum(hi_c, lo_c)
        hival = hi_c - 1
        j_end = jnp.int32(0)
        for k in range(16):
            j_end = j_end + (rv1[k] <= hival).astype(jnp.int32)
        j_cur = car[0]

        def jbody(jj, rc):
            rvj = rsb[pl.ds(jj, 16)]
            lo = jnp.maximum(rvj[0], lo_c)
            hi = jnp.minimum(rvj[1], hi_c)
            rc = rows(lo, hi, base, xb, lb, rc)
            return flush(jj, rc)

        rc = lax.fori_loop(j_cur, j_end, jbody, car[1:])
        rvj = rsb[pl.ds(j_end, 16)]
        lo = jnp.maximum(rvj[0], lo_c)
        hi = jnp.minimum(rvj[1], hi_c)
        rc = rows(lo, hi, base, xb, lb, rc)
        return (j_end,) + rc

    start(0, xb0, lb0, sx0, sl0)
    nc2 = (nch + 1) // 2

    def c2body(c2, car):
        wait(xb0, lb0, sx0, sl0)
        start(2 * c2 + 1, xb1, lb1, sx1, sl1)
        car = process(2 * c2, xb0, lb0, car)
        wait(xb1, lb1, sx1, sl1)
        start(2 * c2 + 2, xb0, lb0, sx0, sl0)
        car = process(2 * c2 + 1, xb1, lb1, car)
        return car

    car = lax.fori_loop(0, nc2, c2body, init_carry)
    wait(xb0, lb0, sx0, sl0)
    flush(car[0], car[1:])
    pltpu.sync_copy(outb, out_hbm.at[pl.ds(base_seg, SEG_PER)])


def kernel(x, batch, W_g1, b_g1, W_g2, b_g2, W_l, b_l):
    del b_g2
    batch3 = batch.astype(jnp.int32).reshape(NBLK, 1, B)
    w2t = W_g2.reshape(1, H)
    bg1 = b_g1.reshape(1, H)
    bl = b_l.reshape(1, DL)

    lg, rs = pl.pallas_call(
        _tc_body,
        grid=(NBLK,),
        in_specs=[
            pl.BlockSpec((1, 1, B), lambda i: (i, 0, 0)),
            pl.BlockSpec((B, D), lambda i: (i, 0)),
            pl.BlockSpec((D, H), lambda i: (0, 0)),
            pl.BlockSpec((1, H), lambda i: (0, 0)),
            pl.BlockSpec((1, H), lambda i: (0, 0)),
            pl.BlockSpec((D, DL), lambda i: (0, 0)),
            pl.BlockSpec((1, DL), lambda i: (0, 0)),
        ],
        out_specs=[
            pl.BlockSpec((B, LGW), lambda i: (i, 0)),
            pl.BlockSpec((RS_PAD,), lambda i: (0,)),
        ],
        out_shape=[
            jax.ShapeDtypeStruct((N, LGW), jnp.float32),
            jax.ShapeDtypeStruct((RS_PAD,), jnp.int32),
        ],
        scratch_shapes=[pltpu.VMEM((RSP, 1), jnp.int32)],
    )(batch3, x, W_g1, bg1, w2t, W_l, bl)

    out = pl.kernel(
        _sc_body,
        out_type=jax.ShapeDtypeStruct((G, OUTW), jnp.float32),
        mesh=plsc.VectorSubcoreMesh(core_axis_name="c", subcore_axis_name="s",
                                    num_cores=2, num_subcores=16),
        scratch_types=[
            pltpu.VMEM((32,), jnp.int32),
            pltpu.VMEM((LD, D), jnp.float32),
            pltpu.VMEM((LD, D), jnp.float32),
            pltpu.VMEM((LD, LGW), jnp.float32),
            pltpu.VMEM((LD, LGW), jnp.float32),
            pltpu.VMEM((SEG_PER, OUTW), jnp.float32),
            pltpu.SemaphoreType.DMA,
            pltpu.SemaphoreType.DMA,
            pltpu.SemaphoreType.DMA,
            pltpu.SemaphoreType.DMA,
        ],
    )(x, lg, rs)
    return out


# speedup vs baseline: 20.9711x; 1.2631x over previous
import jax
import jax.numpy as jnp
from jax import lax
from jax.experimental import pallas as pl
from jax.experimental.pallas import tpu as pltpu
from jax.experimental.pallas import tpu_sc as plsc

N = 100000
D = 128
H = 64
G = 512
DL = 64
B = 4000
NBLK = N // B
RS_PAD = 640
RSW = 72
RSP = 576
AUXW = 8
LD = 488
SEG_PER = 16
SCW = 256
TCW = 192


def _gelu(z):
    return 0.5 * z * (1.0 + lax.erf(z * 0.7071067811865476))


def _rs_body(batch_ref, rs_ref, rs_scr):
    pid = pl.program_id(0)
    b = batch_ref[0, 0, :]
    lo = batch_ref[0, 0, 0]
    hi = batch_ref[0, 0, B - 1]
    wbase = (lo // 8) * 8

    @pl.when(pid == 0)
    def _():
        rs_scr[...] = jnp.zeros((RSP, 1), jnp.int32)

    @pl.when(hi - lo <= RSW - 8)
    def _():
        wg = wbase + lax.broadcasted_iota(jnp.int32, (RSW, B), 0)
        cw = jnp.sum((b[None, :] < wg).astype(jnp.int32), axis=1,
                     keepdims=True)
        wg1 = wbase + lax.broadcasted_iota(jnp.int32, (RSW, 1), 0)
        rs_scr[pl.ds(wbase, RSW), :] += jnp.where(wg1 <= hi, cw, 0)

    @pl.when(hi - lo > RSW - 8)
    def _():
        git = lax.broadcasted_iota(jnp.int32, (RSP, B), 0)
        cf = jnp.sum((b[None, :] < git).astype(jnp.int32), axis=1,
                     keepdims=True)
        git1 = lax.broadcasted_iota(jnp.int32, (RSP, 1), 0)
        rs_scr[...] += jnp.where(git1 <= hi, cf, 0)

    git1 = lax.broadcasted_iota(jnp.int32, (RSP, 1), 0)
    rs_scr[...] += jnp.where(git1 > hi, B, 0)

    @pl.when(pid == NBLK - 1)
    def _():
        rs_ref[...] = jnp.concatenate(
            [rs_scr[...][:G, 0], jnp.full((RS_PAD - G,), N, jnp.int32)])


def _tc_body(batch_ref, x_ref, wg1_ref, bg1_ref, w2t_ref, wl_ref, bl_ref,
             out_ref, ex_scr, ls_scr, aux_scr):
    pid = pl.program_id(0)
    x = x_ref[...]
    h = _gelu(jnp.dot(x, wg1_ref[...], preferred_element_type=jnp.float32)
              + bg1_ref[...])
    gate = jnp.sum(h * w2t_ref[...], axis=1, keepdims=True)
    e = jnp.exp(gate)
    loc = _gelu(jnp.dot(x, wl_ref[...], preferred_element_type=jnp.float32)
                + bl_ref[...])
    exr = x * e
    aux = jnp.concatenate(
        [jnp.ones((B, 1), jnp.float32), e,
         jnp.zeros((B, AUXW - 2), jnp.float32)], axis=1)

    b = batch_ref[0, 0, :]
    lo = batch_ref[0, 0, 0]
    hi = batch_ref[0, 0, B - 1]
    wbase = (lo // 8) * 8

    @pl.when(pid == 0)
    def _():
        ex_scr[...] = jnp.zeros((RSP, D), jnp.float32)
        ls_scr[...] = jnp.zeros((RSP, DL), jnp.float32)
        aux_scr[...] = jnp.zeros((RSP, AUXW), jnp.float32)

    @pl.when(hi - lo <= RSW - 8)
    def _():
        wg = wbase + lax.broadcasted_iota(jnp.int32, (RSW, B), 0)
        oh = (b[None, :] == wg).astype(jnp.float32)
        ex_scr[pl.ds(wbase, RSW), :] += jnp.dot(
            oh, exr, preferred_element_type=jnp.float32)
        ls_scr[pl.ds(wbase, RSW), :] += jnp.dot(
            oh, loc, preferred_element_type=jnp.float32)
        aux_scr[pl.ds(wbase, RSW), :] += jnp.dot(
            oh, aux, preferred_element_type=jnp.float32)

    @pl.when(hi - lo > RSW - 8)
    def _():
        git = lax.broadcasted_iota(jnp.int32, (RSP, B), 0)
        oh = (b[None, :] == git).astype(jnp.float32)
        ex_scr[...] += jnp.dot(oh, exr, preferred_element_type=jnp.float32)
        ls_scr[...] += jnp.dot(oh, loc, preferred_element_type=jnp.float32)
        aux_scr[...] += jnp.dot(oh, aux, preferred_element_type=jnp.float32)

    @pl.when(pid == NBLK - 1)
    def _():
        cnt = aux_scr[...][:G, 0:1]
        esum = aux_scr[...][:G, 1:2]
        den = jnp.where(esum > 0.0, esum, 1.0)
        att = ex_scr[...][:G, :] / den
        locm = ls_scr[...][:G, :] / jnp.maximum(cnt, 1.0)
        out_ref[...] = jnp.concatenate([att, locm], axis=1)


def _sc_body(x_hbm, rs_hbm, out_hbm, rsb, xb0, xb1, outb, sx0, sx1):
    c = lax.axis_index("c")
    s = lax.axis_index("s")
    wid = s * 2 + c
    base_seg = wid * SEG_PER
    pltpu.sync_copy(rs_hbm.at[pl.ds(base_seg, 32)], rsb)
    rv1 = rsb[pl.ds(1, 16)]
    tile_s = rsb[pl.ds(0, 16)][0]
    tile_e = rv1[15]
    ts0 = (tile_s // 8) * 8
    nch = (tile_e - ts0 + LD - 1) // LD

    zero = jnp.zeros((16,), jnp.float32)
    ninf = jnp.full((16,), -jnp.inf, jnp.float32)

    def prefill(j, _):
        for k in range(8):
            outb[j, pl.ds(16 * k, 16)] = zero
        for k in range(8):
            outb[j, pl.ds(128 + 16 * k, 16)] = ninf
        return 0

    lax.fori_loop(0, SEG_PER, prefill, 0)

    def chunk_base(cix):
        return jnp.minimum(ts0 + cix * LD, N - LD)

    def start(cix, xb, sx):
        pltpu.make_async_copy(
            x_hbm.at[pl.ds(chunk_base(cix), LD)], xb, sx).start()

    def wait(xb, sx):
        pltpu.make_async_copy(x_hbm.at[pl.ds(0, LD)], xb, sx).wait()

    init_carry = (jnp.int32(0),) + (zero,) * 8 + (ninf,) * 8

    def rows(lo, hi, base, xb, car):
        def row(r, rc):
            idx = r - base
            xs = [xb[idx, pl.ds(16 * k, 16)] for k in range(8)]
            sx_ = tuple(rc[k] + xs[k] for k in range(8))
            mx_ = tuple(jnp.maximum(rc[8 + k], xs[k]) for k in range(8))
            return sx_ + mx_

        return lax.fori_loop(lo, hi, row, car)

    def flush(jj, cnt, rc):
        cntf = jnp.maximum(cnt.astype(jnp.float32), 1.0)
        inv = 1.0 / jnp.broadcast_to(cntf, (16,))
        for k in range(8):
            outb[jj, pl.ds(16 * k, 16)] = rc[k] * inv
        for k in range(8):
            outb[jj, pl.ds(128 + 16 * k, 16)] = rc[8 + k]
        return (zero,) * 8 + (ninf,) * 8

    def process(cix, xb, car):
        base = chunk_base(cix)
        lo_c = jnp.minimum(jnp.maximum(tile_s, ts0 + cix * LD), tile_e)
        hi_c = jnp.minimum(tile_e, ts0 + (cix + 1) * LD)
        hi_c = jnp.maximum(hi_c, lo_c)
        hival = hi_c - 1
        j_end = jnp.int32(0)
        for k in range(16):
            j_end = j_end + (rv1[k] <= hival).astype(jnp.int32)
        j_cur = car[0]

        def jbody(jj, rc):
            rvj = rsb[pl.ds(jj, 16)]
            lo = jnp.maximum(rvj[0], lo_c)
            hi = jnp.minimum(rvj[1], hi_c)
            rc = rows(lo, hi, base, xb, rc)
            return flush(jj, rvj[1] - rvj[0], rc)

        rc = lax.fori_loop(j_cur, j_end, jbody, car[1:])
        rvj = rsb[pl.ds(j_end, 16)]
        lo = jnp.maximum(rvj[0], lo_c)
        hi = jnp.minimum(rvj[1], hi_c)
        rc = rows(lo, hi, base, xb, rc)
        return (j_end,) + rc

    start(0, xb0, sx0)
    nc2 = (nch + 1) // 2

    def c2body(c2, car):
        wait(xb0, sx0)
        start(2 * c2 + 1, xb1, sx1)
        car = process(2 * c2, xb0, car)
        wait(xb1, sx1)
        start(2 * c2 + 2, xb0, sx0)
        car = process(2 * c2 + 1, xb1, car)
        return car

    car = lax.fori_loop(0, nc2, c2body, init_carry)
    wait(xb0, sx0)
    jf = car[0]
    rvj = rsb[pl.ds(jf, 16)]
    flush(jf, rvj[1] - rvj[0], car[1:])
    pltpu.sync_copy(outb, out_hbm.at[pl.ds(base_seg, SEG_PER)])


def kernel(x, batch, W_g1, b_g1, W_g2, b_g2, W_l, b_l):
    del b_g2
    batch3 = batch.astype(jnp.int32).reshape(NBLK, 1, B)
    w2t = W_g2.reshape(1, H)
    bg1 = b_g1.reshape(1, H)
    bl = b_l.reshape(1, DL)

    rs = pl.pallas_call(
        _rs_body,
        grid=(NBLK,),
        in_specs=[pl.BlockSpec((1, 1, B), lambda i: (i, 0, 0))],
        out_specs=pl.BlockSpec((RS_PAD,), lambda i: (0,)),
        out_shape=jax.ShapeDtypeStruct((RS_PAD,), jnp.int32),
        scratch_shapes=[pltpu.VMEM((RSP, 1), jnp.int32)],
    )(batch3)

    sc_out = pl.kernel(
        _sc_body,
        out_type=jax.ShapeDtypeStruct((G, SCW), jnp.float32),
        mesh=plsc.VectorSubcoreMesh(core_axis_name="c", subcore_axis_name="s",
                                    num_cores=2, num_subcores=16),
        scratch_types=[
            pltpu.VMEM((32,), jnp.int32),
            pltpu.VMEM((LD, D), jnp.float32),
            pltpu.VMEM((LD, D), jnp.float32),
            pltpu.VMEM((SEG_PER, SCW), jnp.float32),
            pltpu.SemaphoreType.DMA,
            pltpu.SemaphoreType.DMA,
        ],
    )(x, rs)

    tc_out = pl.pallas_call(
        _tc_body,
        grid=(NBLK,),
        in_specs=[
            pl.BlockSpec((1, 1, B), lambda i: (i, 0, 0)),
            pl.BlockSpec((B, D), lambda i: (i, 0)),
            pl.BlockSpec((D, H), lambda i: (0, 0)),
            pl.BlockSpec((1, H), lambda i: (0, 0)),
            pl.BlockSpec((1, H), lambda i: (0, 0)),
            pl.BlockSpec((D, DL), lambda i: (0, 0)),
            pl.BlockSpec((1, DL), lambda i: (0, 0)),
        ],
        out_specs=pl.BlockSpec((G, TCW), lambda i: (0, 0)),
        out_shape=jax.ShapeDtypeStruct((G, TCW), jnp.float32),
        scratch_shapes=[
            pltpu.VMEM((RSP, D), jnp.float32),
            pltpu.VMEM((RSP, DL), jnp.float32),
            pltpu.VMEM((RSP, AUXW), jnp.float32),
        ],
    )(batch3, x, W_g1, bg1, w2t, W_l, bl)

    return jnp.concatenate([sc_out, tc_out], axis=1)


# speedup vs baseline: 21.5145x; 1.0259x over previous
import jax
import jax.numpy as jnp
from jax import lax
from jax.experimental import pallas as pl
from jax.experimental.pallas import tpu as pltpu
from jax.experimental.pallas import tpu_sc as plsc

N = 100000
D = 128
H = 64
G = 512
DL = 64
B = 4000
NBLK = N // B
RS_PAD = 640
RSW = 72
RSP = 576
AUXW = 8
LD = 488
SEG_PER = 16
SCW = 256
TCW = 192


def _gelu(z):
    return 0.5 * z * (1.0 + lax.erf(z * 0.7071067811865476))


def _rs_body(batch_ref, rs_ref, rs_scr, co_scr):
    pid = pl.program_id(0)
    b = batch_ref[0, 0, :]
    lo = batch_ref[0, 0, 0]
    hi = batch_ref[0, 0, B - 1]
    wbase = (lo // 8) * 8

    @pl.when(pid == 0)
    def _():
        rs_scr[...] = jnp.zeros((RSP, 1), jnp.int32)
        co_scr[...] = jnp.zeros((RS_PAD // 128, 128), jnp.int32)

    @pl.when(hi - lo <= RSW - 8)
    def _():
        wg = wbase + lax.broadcasted_iota(jnp.int32, (RSW, B), 0)
        cw = jnp.sum((b[None, :] < wg).astype(jnp.int32), axis=1,
                     keepdims=True)
        wg1 = wbase + lax.broadcasted_iota(jnp.int32, (RSW, 1), 0)
        rs_scr[pl.ds(wbase, RSW), :] += jnp.where(wg1 <= hi, cw, 0)

    @pl.when(hi - lo > RSW - 8)
    def _():
        git = lax.broadcasted_iota(jnp.int32, (RSP, B), 0)
        cf = jnp.sum((b[None, :] < git).astype(jnp.int32), axis=1,
                     keepdims=True)
        git1 = lax.broadcasted_iota(jnp.int32, (RSP, 1), 0)
        rs_scr[...] += jnp.where(git1 <= hi, cf, 0)

    gflat = (lax.broadcasted_iota(jnp.int32, (RS_PAD // 128, 128), 0) * 128
             + lax.broadcasted_iota(jnp.int32, (RS_PAD // 128, 128), 1))
    co_scr[...] += jnp.where(gflat > hi, B, 0)

    @pl.when(pid == NBLK - 1)
    def _():
        win = jnp.concatenate(
            [rs_scr[...][:G, 0], jnp.zeros((RS_PAD - G,), jnp.int32)])
        total = win + co_scr[...].reshape(RS_PAD)
        rs_ref[...] = jnp.where(lax.iota(jnp.int32, RS_PAD) >= G, N, total)


def _tc_body(batch_ref, x_ref, wg1_ref, bg1_ref, w2t_ref, wl_ref, bl_ref,
             out_ref, ex_scr, ls_scr, aux_scr):
    pid = pl.program_id(0)
    x = x_ref[...]
    h = _gelu(jnp.dot(x, wg1_ref[...], preferred_element_type=jnp.float32)
              + bg1_ref[...])
    gate = jnp.sum(h * w2t_ref[...], axis=1, keepdims=True)
    e = jnp.exp(gate)
    loc = _gelu(jnp.dot(x, wl_ref[...], preferred_element_type=jnp.float32)
                + bl_ref[...])
    exr = x * e
    aux = jnp.concatenate(
        [jnp.ones((B, 1), jnp.float32), e,
         jnp.zeros((B, AUXW - 2), jnp.float32)], axis=1)

    b = batch_ref[0, 0, :]
    lo = batch_ref[0, 0, 0]
    hi = batch_ref[0, 0, B - 1]
    wbase = (lo // 8) * 8

    @pl.when(pid == 0)
    def _():
        ex_scr[...] = jnp.zeros((RSP, D), jnp.float32)
        ls_scr[...] = jnp.zeros((RSP, DL), jnp.float32)
        aux_scr[...] = jnp.zeros((RSP, AUXW), jnp.float32)

    @pl.when(hi - lo <= RSW - 8)
    def _():
        wg = wbase + lax.broadcasted_iota(jnp.int32, (RSW, B), 0)
        oh = (b[None, :] == wg).astype(jnp.float32)
        ex_scr[pl.ds(wbase, RSW), :] += jnp.dot(
            oh, exr, preferred_element_type=jnp.float32)
        ls_scr[pl.ds(wbase, RSW), :] += jnp.dot(
            oh, loc, preferred_element_type=jnp.float32)
        aux_scr[pl.ds(wbase, RSW), :] += jnp.dot(
            oh, aux, preferred_element_type=jnp.float32)

    @pl.when(hi - lo > RSW - 8)
    def _():
        git = lax.broadcasted_iota(jnp.int32, (RSP, B), 0)
        oh = (b[None, :] == git).astype(jnp.float32)
        ex_scr[...] += jnp.dot(oh, exr, preferred_element_type=jnp.float32)
        ls_scr[...] += jnp.dot(oh, loc, preferred_element_type=jnp.float32)
        aux_scr[...] += jnp.dot(oh, aux, preferred_element_type=jnp.float32)

    @pl.when(pid == NBLK - 1)
    def _():
        cnt = aux_scr[...][:G, 0:1]
        esum = aux_scr[...][:G, 1:2]
        den = jnp.where(esum > 0.0, esum, 1.0)
        att = ex_scr[...][:G, :] / den
        locm = ls_scr[...][:G, :] / jnp.maximum(cnt, 1.0)
        out_ref[...] = jnp.concatenate([att, locm], axis=1)


def _sc_body(x_hbm, rs_hbm, out_hbm, rsb, xb0, xb1, outb, sx0, sx1):
    c = lax.axis_index("c")
    s = lax.axis_index("s")
    wid = s * 2 + c
    base_seg = wid * SEG_PER
    pltpu.sync_copy(rs_hbm.at[pl.ds(base_seg, 32)], rsb)
    rv1 = rsb[pl.ds(1, 16)]
    tile_s = rsb[pl.ds(0, 16)][0]
    tile_e = rv1[15]
    ts0 = (tile_s // 8) * 8
    nch = (tile_e - ts0 + LD - 1) // LD

    zero = jnp.zeros((16,), jnp.float32)
    ninf = jnp.full((16,), -jnp.inf, jnp.float32)

    def prefill(j, _):
        for k in range(8):
            outb[j, pl.ds(16 * k, 16)] = zero
        for k in range(8):
            outb[j, pl.ds(128 + 16 * k, 16)] = ninf
        return 0

    lax.fori_loop(0, SEG_PER, prefill, 0)

    def chunk_base(cix):
        return jnp.minimum(ts0 + cix * LD, N - LD)

    def start(cix, xb, sx):
        pltpu.make_async_copy(
            x_hbm.at[pl.ds(chunk_base(cix), LD)], xb, sx).start()

    def wait(xb, sx):
        pltpu.make_async_copy(x_hbm.at[pl.ds(0, LD)], xb, sx).wait()

    init_carry = (jnp.int32(0),) + (zero,) * 8 + (ninf,) * 8

    def rows(lo, hi, base, xb, car):
        def row(r, rc):
            idx = r - base
            xs = [xb[idx, pl.ds(16 * k, 16)] for k in range(8)]
            sx_ = tuple(rc[k] + xs[k] for k in range(8))
            mx_ = tuple(jnp.maximum(rc[8 + k], xs[k]) for k in range(8))
            return sx_ + mx_

        return lax.fori_loop(lo, hi, row, car)

    def flush(jj, cnt, rc):
        cntf = jnp.maximum(cnt.astype(jnp.float32), 1.0)
        inv = 1.0 / jnp.broadcast_to(cntf, (16,))
        for k in range(8):
            outb[jj, pl.ds(16 * k, 16)] = rc[k] * inv
        for k in range(8):
            outb[jj, pl.ds(128 + 16 * k, 16)] = rc[8 + k]
        return (zero,) * 8 + (ninf,) * 8

    def process(cix, xb, car):
        base = chunk_base(cix)
        lo_c = jnp.minimum(jnp.maximum(tile_s, ts0 + cix * LD), tile_e)
        hi_c = jnp.minimum(tile_e, ts0 + (cix + 1) * LD)
        hi_c = jnp.maximum(hi_c, lo_c)
        hival = hi_c - 1
        j_end = jnp.int32(0)
        for k in range(16):
            j_end = j_end + (rv1[k] <= hival).astype(jnp.int32)
        j_cur = car[0]

        def jbody(jj, rc):
            rvj = rsb[pl.ds(jj, 16)]
            lo = jnp.maximum(rvj[0], lo_c)
            hi = jnp.minimum(rvj[1], hi_c)
            rc = rows(lo, hi, base, xb, rc)
            return flush(jj, rvj[1] - rvj[0], rc)

        rc = lax.fori_loop(j_cur, j_end, jbody, car[1:])
        rvj = rsb[pl.ds(j_end, 16)]
        lo = jnp.maximum(rvj[0], lo_c)
        hi = jnp.minimum(rvj[1], hi_c)
        rc = rows(lo, hi, base, xb, rc)
        return (j_end,) + rc

    start(0, xb0, sx0)
    nc2 = (nch + 1) // 2

    def c2body(c2, car):
        wait(xb0, sx0)
        start(2 * c2 + 1, xb1, sx1)
        car = process(2 * c2, xb0, car)
        wait(xb1, sx1)
        start(2 * c2 + 2, xb0, sx0)
        car = process(2 * c2 + 1, xb1, car)
        return car

    car = lax.fori_loop(0, nc2, c2body, init_carry)
    wait(xb0, sx0)
    jf = car[0]
    rvj = rsb[pl.ds(jf, 16)]
    flush(jf, rvj[1] - rvj[0], car[1:])
    pltpu.sync_copy(outb, out_hbm.at[pl.ds(base_seg, SEG_PER)])


def kernel(x, batch, W_g1, b_g1, W_g2, b_g2, W_l, b_l):
    del b_g2
    batch3 = batch.astype(jnp.int32).reshape(NBLK, 1, B)
    w2t = W_g2.reshape(1, H)
    bg1 = b_g1.reshape(1, H)
    bl = b_l.reshape(1, DL)

    rs = pl.pallas_call(
        _rs_body,
        grid=(NBLK,),
        in_specs=[pl.BlockSpec((1, 1, B), lambda i: (i, 0, 0))],
        out_specs=pl.BlockSpec((RS_PAD,), lambda i: (0,)),
        out_shape=jax.ShapeDtypeStruct((RS_PAD,), jnp.int32),
        scratch_shapes=[pltpu.VMEM((RSP, 1), jnp.int32),
                        pltpu.VMEM((RS_PAD // 128, 128), jnp.int32)],
    )(batch3)

    sc_out = pl.kernel(
        _sc_body,
        out_type=jax.ShapeDtypeStruct((G, SCW), jnp.float32),
        mesh=plsc.VectorSubcoreMesh(core_axis_name="c", subcore_axis_name="s",
                                    num_cores=2, num_subcores=16),
        scratch_types=[
            pltpu.VMEM((32,), jnp.int32),
            pltpu.VMEM((LD, D), jnp.float32),
            pltpu.VMEM((LD, D), jnp.float32),
            pltpu.VMEM((SEG_PER, SCW), jnp.float32),
            pltpu.SemaphoreType.DMA,
            pltpu.SemaphoreType.DMA,
        ],
    )(x, rs)

    tc_out = pl.pallas_call(
        _tc_body,
        grid=(NBLK,),
        in_specs=[
            pl.BlockSpec((1, 1, B), lambda i: (i, 0, 0)),
            pl.BlockSpec((B, D), lambda i: (i, 0)),
            pl.BlockSpec((D, H), lambda i: (0, 0)),
            pl.BlockSpec((1, H), lambda i: (0, 0)),
            pl.BlockSpec((1, H), lambda i: (0, 0)),
            pl.BlockSpec((D, DL), lambda i: (0, 0)),
            pl.BlockSpec((1, DL), lambda i: (0, 0)),
        ],
        out_specs=pl.BlockSpec((G, TCW), lambda i: (0, 0)),
        out_shape=jax.ShapeDtypeStruct((G, TCW), jnp.float32),
        scratch_shapes=[
            pltpu.VMEM((RSP, D), jnp.float32),
            pltpu.VMEM((RSP, DL), jnp.float32),
            pltpu.VMEM((RSP, AUXW), jnp.float32),
        ],
    )(batch3, x, W_g1, bg1, w2t, W_l, bl)

    return jnp.concatenate([sc_out, tc_out], axis=1)


# speedup vs baseline: 23.0647x; 1.0721x over previous
import jax
import jax.numpy as jnp
from jax import lax
from jax.experimental import pallas as pl
from jax.experimental.pallas import tpu as pltpu
from jax.experimental.pallas import tpu_sc as plsc

N = 100000
D = 128
H = 64
G = 512
DL = 64
B = 4000
NBLK = N // B
RS_PAD = 640
RSW = 48
RSP = 576
LAW = 72
LD = 488
SEG_PER = 16
SCW = 256
TCW = 192


def _gelu(z):
    return 0.5 * z * (1.0 + lax.erf(z * 0.7071067811865476))


def _rs_body(batch_ref, rs_ref, rs_scr, co_scr):
    pid = pl.program_id(0)
    b = batch_ref[0, 0, :]
    lo = batch_ref[0, 0, 0]
    hi = batch_ref[0, 0, B - 1]
    wbase = (lo // 8) * 8

    @pl.when(pid == 0)
    def _():
        rs_scr[...] = jnp.zeros((RSP, 1), jnp.int32)
        co_scr[...] = jnp.zeros((RS_PAD // 128, 128), jnp.int32)

    @pl.when(hi - lo <= RSW - 8)
    def _():
        wg = wbase + lax.broadcasted_iota(jnp.int32, (RSW, B), 0)
        cw = jnp.sum((b[None, :] < wg).astype(jnp.int32), axis=1,
                     keepdims=True)
        wg1 = wbase + lax.broadcasted_iota(jnp.int32, (RSW, 1), 0)
        rs_scr[pl.ds(wbase, RSW), :] += jnp.where(wg1 <= hi, cw, 0)

    @pl.when(hi - lo > RSW - 8)
    def _():
        git = lax.broadcasted_iota(jnp.int32, (RSP, B), 0)
        cf = jnp.sum((b[None, :] < git).astype(jnp.int32), axis=1,
                     keepdims=True)
        git1 = lax.broadcasted_iota(jnp.int32, (RSP, 1), 0)
        rs_scr[...] += jnp.where(git1 <= hi, cf, 0)

    gflat = (lax.broadcasted_iota(jnp.int32, (RS_PAD // 128, 128), 0) * 128
             + lax.broadcasted_iota(jnp.int32, (RS_PAD // 128, 128), 1))
    co_scr[...] += jnp.where(gflat > hi, B, 0)

    @pl.when(pid == NBLK - 1)
    def _():
        win = jnp.concatenate(
            [rs_scr[...][:G, 0], jnp.zeros((RS_PAD - G,), jnp.int32)])
        total = win + co_scr[...].reshape(RS_PAD)
        rs_ref[...] = jnp.where(lax.iota(jnp.int32, RS_PAD) >= G, N, total)


def _tc_body(batch_ref, x_ref, wcat_ref, bcat_ref, w2t_ref,
             out_ref, ex_scr, la_scr):
    pid = pl.program_id(0)
    x = x_ref[...]
    h = _gelu(jnp.dot(x, wcat_ref[...], preferred_element_type=jnp.float32)
              + bcat_ref[...])
    gate = jnp.sum(h[:, :H] * w2t_ref[...], axis=1, keepdims=True)
    e = jnp.exp(gate)
    exr = x * e
    la = jnp.concatenate(
        [h[:, H:], jnp.ones((B, 1), jnp.float32), e,
         jnp.zeros((B, LAW - DL - 2), jnp.float32)], axis=1)

    b = batch_ref[0, 0, :]
    lo = batch_ref[0, 0, 0]
    hi = batch_ref[0, 0, B - 1]
    wbase = (lo // 8) * 8

    @pl.when(pid == 0)
    def _():
        ex_scr[...] = jnp.zeros((RSP, D), jnp.float32)
        la_scr[...] = jnp.zeros((RSP, LAW), jnp.float32)

    @pl.when(hi - lo <= RSW - 8)
    def _():
        wg = wbase + lax.broadcasted_iota(jnp.int32, (RSW, B), 0)
        oh = (b[None, :] == wg).astype(jnp.float32)
        ex_scr[pl.ds(wbase, RSW), :] += jnp.dot(
            oh, exr, preferred_element_type=jnp.float32)
        la_scr[pl.ds(wbase, RSW), :] += jnp.dot(
            oh, la, preferred_element_type=jnp.float32)

    @pl.when(hi - lo > RSW - 8)
    def _():
        git = lax.broadcasted_iota(jnp.int32, (RSP, B), 0)
        oh = (b[None, :] == git).astype(jnp.float32)
        ex_scr[...] += jnp.dot(oh, exr, preferred_element_type=jnp.float32)
        la_scr[...] += jnp.dot(oh, la, preferred_element_type=jnp.float32)

    @pl.when(pid == NBLK - 1)
    def _():
        cnt = la_scr[...][:G, DL:DL + 1]
        esum = la_scr[...][:G, DL + 1:DL + 2]
        den = jnp.where(esum > 0.0, esum, 1.0)
        att = ex_scr[...][:G, :] / den
        locm = la_scr[...][:G, :DL] / jnp.maximum(cnt, 1.0)
        out_ref[...] = jnp.concatenate([att, locm], axis=1)


def _sc_body(x_hbm, rs_hbm, out_hbm, rsb, xb0, xb1, outb, sx0, sx1):
    c = lax.axis_index("c")
    s = lax.axis_index("s")
    wid = s * 2 + c
    base_seg = wid * SEG_PER
    pltpu.sync_copy(rs_hbm.at[pl.ds(base_seg, 32)], rsb)
    rv1 = rsb[pl.ds(1, 16)]
    tile_s = rsb[pl.ds(0, 16)][0]
    tile_e = rv1[15]
    ts0 = (tile_s // 8) * 8
    nch = (tile_e - ts0 + LD - 1) // LD

    zero = jnp.zeros((16,), jnp.float32)
    ninf = jnp.full((16,), -jnp.inf, jnp.float32)

    def prefill(j, _):
        for k in range(8):
            outb[j, pl.ds(16 * k, 16)] = zero
        for k in range(8):
            outb[j, pl.ds(128 + 16 * k, 16)] = ninf
        return 0

    lax.fori_loop(0, SEG_PER, prefill, 0)

    def chunk_base(cix):
        return jnp.minimum(ts0 + cix * LD, N - LD)

    def start(cix, xb, sx):
        pltpu.make_async_copy(
            x_hbm.at[pl.ds(chunk_base(cix), LD)], xb, sx).start()

    def wait(xb, sx):
        pltpu.make_async_copy(x_hbm.at[pl.ds(0, LD)], xb, sx).wait()

    init_carry = (jnp.int32(0),) + (zero,) * 8 + (ninf,) * 8

    def rows(lo, hi, base, xb, car):
        def row(r, rc):
            idx = r - base
            xs = [xb[idx, pl.ds(16 * k, 16)] for k in range(8)]
            sx_ = tuple(rc[k] + xs[k] for k in range(8))
            mx_ = tuple(jnp.maximum(rc[8 + k], xs[k]) for k in range(8))
            return sx_ + mx_

        return lax.fori_loop(lo, hi, row, car)

    def flush(jj, cnt, rc):
        cntf = jnp.maximum(cnt.astype(jnp.float32), 1.0)
        inv = 1.0 / jnp.broadcast_to(cntf, (16,))
        for k in range(8):
            outb[jj, pl.ds(16 * k, 16)] = rc[k] * inv
        for k in range(8):
            outb[jj, pl.ds(128 + 16 * k, 16)] = rc[8 + k]
        return (zero,) * 8 + (ninf,) * 8

    def process(cix, xb, car):
        base = chunk_base(cix)
        lo_c = jnp.minimum(jnp.maximum(tile_s, ts0 + cix * LD), tile_e)
        hi_c = jnp.minimum(tile_e, ts0 + (cix + 1) * LD)
        hi_c = jnp.maximum(hi_c, lo_c)
        hival = hi_c - 1
        j_end = jnp.int32(0)
        for k in range(16):
            j_end = j_end + (rv1[k] <= hival).astype(jnp.int32)
        j_cur = car[0]

        def jbody(jj, rc):
            rvj = rsb[pl.ds(jj, 16)]
            lo = jnp.maximum(rvj[0], lo_c)
            hi = jnp.minimum(rvj[1], hi_c)
            rc = rows(lo, hi, base, xb, rc)
            return flush(jj, rvj[1] - rvj[0], rc)

        rc = lax.fori_loop(j_cur, j_end, jbody, car[1:])
        rvj = rsb[pl.ds(j_end, 16)]
        lo = jnp.maximum(rvj[0], lo_c)
        hi = jnp.minimum(rvj[1], hi_c)
        rc = rows(lo, hi, base, xb, rc)
        return (j_end,) + rc

    start(0, xb0, sx0)
    nc2 = (nch + 1) // 2

    def c2body(c2, car):
        wait(xb0, sx0)
        start(2 * c2 + 1, xb1, sx1)
        car = process(2 * c2, xb0, car)
        wait(xb1, sx1)
        start(2 * c2 + 2, xb0, sx0)
        car = process(2 * c2 + 1, xb1, car)
        return car

    car = lax.fori_loop(0, nc2, c2body, init_carry)
    wait(xb0, sx0)
    jf = car[0]
    rvj = rsb[pl.ds(jf, 16)]
    flush(jf, rvj[1] - rvj[0], car[1:])
    pltpu.sync_copy(outb, out_hbm.at[pl.ds(base_seg, SEG_PER)])


def kernel(x, batch, W_g1, b_g1, W_g2, b_g2, W_l, b_l):
    del b_g2
    batch3 = batch.astype(jnp.int32).reshape(NBLK, 1, B)
    w2t = W_g2.reshape(1, H)
    wcat = jnp.concatenate([W_g1, W_l], axis=1)
    bcat = jnp.concatenate([b_g1, b_l]).reshape(1, H + DL)

    rs = pl.pallas_call(
        _rs_body,
        grid=(NBLK,),
        in_specs=[pl.BlockSpec((1, 1, B), lambda i: (i, 0, 0))],
        out_specs=pl.BlockSpec((RS_PAD,), lambda i: (0,)),
        out_shape=jax.ShapeDtypeStruct((RS_PAD,), jnp.int32),
        scratch_shapes=[pltpu.VMEM((RSP, 1), jnp.int32),
                        pltpu.VMEM((RS_PAD // 128, 128), jnp.int32)],
    )(batch3)

    sc_out = pl.kernel(
        _sc_body,
        out_type=jax.ShapeDtypeStruct((G, SCW), jnp.float32),
        mesh=plsc.VectorSubcoreMesh(core_axis_name="c", subcore_axis_name="s",
                                    num_cores=2, num_subcores=16),
        scratch_types=[
            pltpu.VMEM((32,), jnp.int32),
            pltpu.VMEM((LD, D), jnp.float32),
            pltpu.VMEM((LD, D), jnp.float32),
            pltpu.VMEM((SEG_PER, SCW), jnp.float32),
            pltpu.SemaphoreType.DMA,
            pltpu.SemaphoreType.DMA,
        ],
    )(x, rs)

    tc_out = pl.pallas_call(
        _tc_body,
        grid=(NBLK,),
        in_specs=[
            pl.BlockSpec((1, 1, B), lambda i: (i, 0, 0)),
            pl.BlockSpec((B, D), lambda i: (i, 0)),
            pl.BlockSpec((D, H + DL), lambda i: (0, 0)),
            pl.BlockSpec((1, H + DL), lambda i: (0, 0)),
            pl.BlockSpec((1, H), lambda i: (0, 0)),
        ],
        out_specs=pl.BlockSpec((G, TCW), lambda i: (0, 0)),
        out_shape=jax.ShapeDtypeStruct((G, TCW), jnp.float32),
        scratch_shapes=[
            pltpu.VMEM((RSP, D), jnp.float32),
            pltpu.VMEM((RSP, LAW), jnp.float32),
        ],
    )(batch3, x, wcat, bcat, w2t)

    return jnp.concatenate([sc_out, tc_out], axis=1)
